# Initial kernel scaffold; baseline (speedup 1.0000x reference)
import jax
import jax.numpy as jnp
from jax.experimental import pallas as pl


def kernel(x, codebooks, W_in, b_in, W_out, b_out):
    raise NotImplementedError("write your pallas kernel here")


# speedup vs baseline: 5.4288x; 5.4288x over previous
import functools

import jax
import jax.numpy as jnp
from jax import lax
from jax.experimental import pallas as pl
from jax.experimental.pallas import tpu as pltpu
from jax.experimental.pallas import tpu_sc as plsc

L = 3
K = 16
DIMS = 8
DIM_Z = 4
N = 2097152

NC = 2
NS = 16
NW = NC * NS
ROWS_PER_W = N // NW
BLK = 2048
NBLK = ROWS_PER_W // BLK
G = BLK // 16

CB_OFF = 0
CBT_OFF = CB_OFF + L * DIM_Z * K
CN_OFF = CBT_OFF + L * DIM_Z * K
WIN_OFF = CN_OFF + L * K
BIN_OFF = WIN_OFF + DIMS * DIM_Z
WOUT_OFF = BIN_OFF + DIM_Z
BOUT_OFF = WOUT_OFF + DIM_Z * DIMS
PARAM_LEN = 512


def _bf16_round(v):
    i = plsc.bitcast(v, jnp.int32)
    r = (i + jnp.int32(0x7FFF) + ((i >> 16) & jnp.int32(1))) & jnp.int32(-65536)
    return plsc.bitcast(r, jnp.float32)


def _take16(vec, idx):
    return lax.gather(
        vec,
        idx[:, None],
        dimension_numbers=lax.GatherDimensionNumbers(
            offset_dims=(),
            collapsed_slice_dims=(0,),
            start_index_map=(0,),
        ),
        slice_sizes=(1,),
        mode=lax.GatherScatterMode.PROMISE_IN_BOUNDS,
    )


def _vq_body(x_hbm, par_hbm, out_hbm, xv, ov, parv):
    wid = lax.axis_index("s") * NC + lax.axis_index("c")

    pltpu.sync_copy(par_hbm, parv)

    cbv = [[parv[pl.ds(CB_OFF + (l * DIM_Z + j) * K, 16)] for j in range(DIM_Z)]
           for l in range(L)]
    cbtv = [[parv[pl.ds(CBT_OFF + (l * DIM_Z + j) * K, 16)]
             for j in range(DIM_Z)] for l in range(L)]
    cnv = [parv[pl.ds(CN_OFF + l * K, 16)] for l in range(L)]
    cn = [[cnv[l][k] for k in range(K)] for l in range(L)]
    cbs = [[[cbtv[l][j][k] for j in range(DIM_Z)]
            for k in range(K)] for l in range(L)]
    tailv = [parv[pl.ds(WIN_OFF + c * 16, 16)]
             for c in range((PARAM_LEN - WIN_OFF) // 16)]

    def _scal(i):
        return tailv[(i - WIN_OFF) // 16][(i - WIN_OFF) % 16]

    win = [[_scal(WIN_OFF + d * DIM_Z + j) for j in range(DIM_Z)]
           for d in range(DIMS)]
    bin_ = [_scal(BIN_OFF + j) for j in range(DIM_Z)]
    wout = [[_scal(WOUT_OFF + j * DIMS + d) for d in range(DIMS)]
            for j in range(DIM_Z)]
    bout = [_scal(BOUT_OFF + d) for d in range(DIMS)]

    lane8 = lax.iota(jnp.int32, 16) * DIMS

    def group_body(g, carry):
        rb = lane8 + g * (16 * DIMS)
        xs = [_bf16_round(plsc.load_gather(xv, [rb + d])) for d in range(DIMS)]

        z = []
        for j in range(DIM_Z):
            acc = bin_[j] + xs[0] * win[0][j]
            for d in range(1, DIMS):
                acc = acc + xs[d] * win[d][j]
            z.append(acc)

        r = list(z)
        qs = None
        for l in range(L):
            rt = [_bf16_round(r[j]) for j in range(DIM_Z)]
            best = None
            bidx = None
            for k in range(K):
                s = cn[l][k] - rt[0] * cbs[l][k][0]
                for j in range(1, DIM_Z):
                    s = s - rt[j] * cbs[l][k][j]
                if best is None:
                    best = s
                    bidx = jnp.zeros((16,), jnp.int32)
                else:
                    m = s < best
                    best = jnp.where(m, s, best)
                    bidx = jnp.where(m, jnp.int32(k), bidx)
            q = [_take16(cbv[l][j], bidx) for j in range(DIM_Z)]
            r = [r[j] - q[j] for j in range(DIM_Z)]
            qs = q if qs is None else [qs[j] + q[j] for j in range(DIM_Z)]

        for d in range(DIMS):
            acc = bout[d] + qs[0] * wout[0][d]
            for j in range(1, DIM_Z):
                acc = acc + qs[j] * wout[j][d]
            plsc.store_scatter(ov, [rb + d], acc)
        return carry

    def blk_body(b, carry):
        off = pl.multiple_of((wid * ROWS_PER_W + b * BLK) * DIMS,
                             BLK * DIMS)
        pltpu.sync_copy(x_hbm.at[pl.ds(off, BLK * DIMS)], xv)
        lax.fori_loop(0, G, group_body, 0, unroll=False)
        pltpu.sync_copy(ov, out_hbm.at[pl.ds(off, BLK * DIMS)])
        return carry

    lax.fori_loop(0, NBLK, blk_body, 0, unroll=False)


@jax.jit
def _vq(x_flat, params):
    mesh = plsc.VectorSubcoreMesh(core_axis_name="c", subcore_axis_name="s")
    f = functools.partial(
        pl.kernel,
        mesh=mesh,
        out_type=jax.ShapeDtypeStruct((N * DIMS,), jnp.float32),
        scratch_types=[
            pltpu.VMEM((BLK * DIMS,), jnp.float32),
            pltpu.VMEM((BLK * DIMS,), jnp.float32),
            pltpu.VMEM((PARAM_LEN,), jnp.float32),
        ],
        compiler_params=pltpu.CompilerParams(needs_layout_passes=False),
    )(_vq_body)
    return f(x_flat, params)


def _bf16_round_host(a):
    i = lax.bitcast_convert_type(a, jnp.int32)
    r = (i + jnp.int32(0x7FFF) + ((i >> 16) & jnp.int32(1))) & jnp.int32(-65536)
    return lax.bitcast_convert_type(r, jnp.float32)


def kernel(x, codebooks, W_in, b_in, W_out, b_out):
    cb_t = jnp.transpose(codebooks, (0, 2, 1))
    cbt_bf = _bf16_round_host(cb_t)
    cn = 0.5 * jnp.sum(codebooks * codebooks, axis=-1)
    win_bf = _bf16_round_host(W_in)
    params = jnp.concatenate([
        cb_t.reshape(-1),
        cbt_bf.reshape(-1),
        cn.reshape(-1),
        win_bf.reshape(-1),
        b_in.reshape(-1),
        W_out.reshape(-1),
        b_out.reshape(-1),
        jnp.zeros((PARAM_LEN - BOUT_OFF - DIMS,), jnp.float32),
    ])
    out_flat = _vq(x.reshape(-1), params)
    return out_flat.reshape(N, DIMS)


# speedup vs baseline: 8.9030x; 1.6400x over previous
import functools

import jax
import jax.numpy as jnp
from jax import lax
from jax.experimental import pallas as pl
from jax.experimental.pallas import tpu as pltpu
from jax.experimental.pallas import tpu_sc as plsc

L = 3
K = 16
DIMS = 8
DIM_Z = 4
N = 2097152

NC = 2
NS = 16
NW = NC * NS
ROWS_PER_W = N // NW
BLK = 2048
NBLK = ROWS_PER_W // BLK
G = BLK // 16

CB_OFF = 0
CBT_OFF = CB_OFF + L * DIM_Z * K
P_OFF = CBT_OFF + L * DIM_Z * K
CNS_OFF = P_OFF + L * DIMS * K
WINS_OFF = CNS_OFF + L * K * 16
BINS_OFF = WINS_OFF + DIMS * DIM_Z * 16
PARAM_LEN = BINS_OFF + DIM_Z * 16


def _bf16_round(v):
    i = plsc.bitcast(v, jnp.int32)
    r = (i + jnp.int32(0x8000)) & jnp.int32(-65536)
    return plsc.bitcast(r, jnp.float32)


def _vq_body(x_hbm, par_hbm, out_hbm, xv, ov, parv):
    wid = lax.axis_index("s") * NC + lax.axis_index("c")

    pltpu.sync_copy(par_hbm, parv)

    lane8 = lax.iota(jnp.int32, 16) * DIMS

    def group_body(g, carry):
        cbtv = [[parv[pl.ds(CBT_OFF + (l * DIM_Z + j) * K, 16)]
                 for j in range(DIM_Z)] for l in range(L)]
        idx0 = lane8 + g * (16 * DIMS)
        idxd = [idx0] + [idx0 + d for d in range(1, DIMS)]
        xs = [_bf16_round(plsc.load_gather(xv, [idxd[d]]))
              for d in range(DIMS)]

        z = []
        for j in range(DIM_Z):
            acc = parv[pl.ds(BINS_OFF + j * 16, 16)]
            for d in range(DIMS):
                acc = acc + xs[d] * parv[pl.ds(WINS_OFF + (d * DIM_Z + j) * 16, 16)]
            z.append(acc)

        r = list(z)
        bidx = []
        for l in range(L):
            rt = [_bf16_round(r[j]) for j in range(DIM_Z)]
            best = None
            for k in range(K):
                s = parv[pl.ds(CNS_OFF + (l * K + k) * 16, 16)]
                for j in range(DIM_Z):
                    s = s - rt[j] * cbtv[l][j][k]
                si = plsc.bitcast(s, jnp.int32) & jnp.int32(-16)
                if k:
                    si = si | jnp.int32(k)
                sp = plsc.bitcast(si, jnp.float32)
                best = sp if best is None else jnp.minimum(best, sp)
            bi = plsc.bitcast(best, jnp.int32) & jnp.int32(15)
            bidx.append(bi)
            if l < L - 1:
                q = [plsc.load_gather(
                        parv, [bi + (CB_OFF + (l * DIM_Z + j) * K)])
                     for j in range(DIM_Z)]
                r = [r[j] - q[j] for j in range(DIM_Z)]

        for d in range(DIMS):
            acc = plsc.load_gather(
                parv, [bidx[0] + (P_OFF + d * K)])
            for l in range(1, L):
                acc = acc + plsc.load_gather(
                    parv, [bidx[l] + (P_OFF + (l * DIMS + d) * K)])
            plsc.store_scatter(ov, [idxd[d]], acc)
        return carry

    def blk_body(b, carry):
        off = pl.multiple_of((wid * ROWS_PER_W + b * BLK) * DIMS,
                             BLK * DIMS)
        pltpu.sync_copy(x_hbm.at[pl.ds(off, BLK * DIMS)], xv)
        lax.fori_loop(0, G, group_body, 0, unroll=False)
        pltpu.sync_copy(ov, out_hbm.at[pl.ds(off, BLK * DIMS)])
        return carry

    lax.fori_loop(0, NBLK, blk_body, 0, unroll=False)


@jax.jit
def _vq(x_flat, params):
    mesh = plsc.VectorSubcoreMesh(core_axis_name="c", subcore_axis_name="s")
    f = functools.partial(
        pl.kernel,
        mesh=mesh,
        out_type=jax.ShapeDtypeStruct((N * DIMS,), jnp.float32),
        scratch_types=[
            pltpu.VMEM((BLK * DIMS,), jnp.float32),
            pltpu.VMEM((BLK * DIMS,), jnp.float32),
            pltpu.VMEM((PARAM_LEN,), jnp.float32),
        ],
        compiler_params=pltpu.CompilerParams(needs_layout_passes=False),
    )(_vq_body)
    return f(x_flat, params)


def _bf16_round_host(a):
    i = lax.bitcast_convert_type(a, jnp.int32)
    r = (i + jnp.int32(0x7FFF) + ((i >> 16) & jnp.int32(1))) & jnp.int32(-65536)
    return lax.bitcast_convert_type(r, jnp.float32)


def kernel(x, codebooks, W_in, b_in, W_out, b_out):
    cb_t = jnp.transpose(codebooks, (0, 2, 1))
    cbt_bf = _bf16_round_host(cb_t)
    cn = 0.5 * jnp.sum(codebooks * codebooks, axis=-1)
    win_bf = _bf16_round_host(W_in)
    pcb = jnp.einsum("lkj,jd->ldk", codebooks, _bf16_round_host(W_out))
    pcb = pcb.at[0].add(b_out[:, None])
    cns = jnp.broadcast_to(cn.reshape(L * K, 1), (L * K, 16))
    wins = jnp.broadcast_to(win_bf.reshape(DIMS * DIM_Z, 1),
                            (DIMS * DIM_Z, 16))
    bins = jnp.broadcast_to(b_in.reshape(DIM_Z, 1), (DIM_Z, 16))
    params = jnp.concatenate([
        cb_t.reshape(-1),
        cbt_bf.reshape(-1),
        pcb.reshape(-1),
        cns.reshape(-1),
        wins.reshape(-1),
        bins.reshape(-1),
    ])
    out_flat = _vq(x.reshape(-1), params)
    return out_flat.reshape(N, DIMS)
